# baseline (speedup 1.0000x reference)
import jax
import jax.numpy as jnp
from jax.experimental import pallas as pl
from jax.experimental.pallas import tpu as pltpu


def _copy_row(x_hbm, o_ref, sem):
    c = pltpu.make_async_copy(x_hbm.at[pl.ds(0, 1)], o_ref, sem)
    c.start()
    c.wait()


def kernel(x):
    return pl.pallas_call(
        _copy_row,
        out_shape=jax.ShapeDtypeStruct((1, 128), jnp.float32),
        in_specs=[pl.BlockSpec(memory_space=pltpu.HBM)],
        out_specs=pl.BlockSpec(memory_space=pltpu.VMEM),
        scratch_shapes=[pltpu.SemaphoreType.DMA],
    )(x)


# speedup vs baseline: 1.2643x; 1.2643x over previous
import jax
import jax.numpy as jnp
from jax.experimental import pallas as pl
from jax.experimental.pallas import tpu as pltpu


def _copy_row(x_hbm, o_hbm, sem):
    pltpu.make_async_copy(x_hbm.at[pl.ds(0, 1)], o_hbm, sem).start()
    pltpu.make_async_copy(x_hbm.at[pl.ds(0, 1)], o_hbm, sem).wait()


def kernel(x):
    return pl.pallas_call(
        _copy_row,
        out_shape=jax.ShapeDtypeStruct((1, 128), jnp.float32),
        in_specs=[pl.BlockSpec(memory_space=pltpu.HBM)],
        out_specs=pl.BlockSpec(memory_space=pltpu.HBM),
        scratch_shapes=[pltpu.SemaphoreType.DMA],
    )(x)
